# Initial kernel scaffold; baseline (speedup 1.0000x reference)
import jax
import jax.numpy as jnp
from jax.experimental import pallas as pl


def kernel(x, active_block_indices, bin_counts, W, b, gamma, beta, running_mean, running_var):
    raise NotImplementedError("write your pallas kernel here")


# speedup vs baseline: 20.4260x; 20.4260x over previous
import jax
import jax.numpy as jnp
from jax.experimental import pallas as pl
from jax.experimental.pallas import tpu as pltpu

_BS = 16
_C = 32
_ROW = 1024 * _C
_BC = _BS * _C
_EPS = 1e-3


def _copy_body(x_ref, o_ref):
    o_ref[...] = x_ref[...]


def _conv_body(sidx_ref, yany_ref, x_ref, a_ref, t_ref, o_ref):
    del sidx_ref, yany_ref
    p = x_ref[...]
    z = jnp.zeros((1, _BC), p.dtype)
    p_up = jnp.concatenate([z, p[:-1, :]], axis=0)
    p_dn = jnp.concatenate([p[1:, :], z], axis=0)
    pc = jnp.concatenate([p_up, p, p_dn], axis=1)
    q = jnp.dot(pc, a_ref[...], preferred_element_type=jnp.float32)
    o_ref[...] = jnp.maximum(q + t_ref[...], 0.0)


def kernel(x, active_block_indices, bin_counts, W, b, gamma, beta,
           running_mean, running_var):
    del bin_counts
    N, H, Wd, C = x.shape
    gh = H // _BS
    gw = Wd // _BS
    nact = active_block_indices.shape[0]

    x2d = x.reshape(H, Wd * C)

    by = active_block_indices[:, 1] % gh
    bx = active_block_indices[:, 2] % gw
    sidx = jnp.stack([by, bx]).astype(jnp.int32)

    s = gamma * jax.lax.rsqrt(running_var + _EPS)
    t = (b - running_mean) * s + beta
    wts = jnp.transpose(W, (2, 3, 1, 0)) * s

    a_rows = []
    for dy in range(3):
        a = jnp.zeros((_BC, _BC), jnp.float32)
        for dx in range(3):
            a = a + jnp.kron(jnp.eye(_BS, k=1 - dx, dtype=jnp.float32),
                             wts[dy, dx])
        a_rows.append(a)
    a_all = jnp.concatenate(a_rows, axis=0)
    t_row = jnp.tile(t, _BS).reshape(1, _BC)

    ycopy = pl.pallas_call(
        _copy_body,
        grid=(H // _BS,),
        in_specs=[pl.BlockSpec((_BS, _ROW), lambda i: (i, 0))],
        out_specs=pl.BlockSpec((_BS, _ROW), lambda i: (i, 0)),
        out_shape=jax.ShapeDtypeStruct((H, Wd * C), jnp.float32),
    )(x2d)

    grid_spec = pltpu.PrefetchScalarGridSpec(
        num_scalar_prefetch=1,
        grid=(nact,),
        in_specs=[
            pl.BlockSpec(memory_space=pl.ANY),
            pl.BlockSpec((_BS, _BC), lambda i, s: (s[0, i], s[1, i])),
            pl.BlockSpec((3 * _BC, _BC), lambda i, s: (0, 0)),
            pl.BlockSpec((1, _BC), lambda i, s: (0, 0)),
        ],
        out_specs=pl.BlockSpec((_BS, _BC), lambda i, s: (s[0, i], s[1, i])),
    )
    y2d = pl.pallas_call(
        _conv_body,
        grid_spec=grid_spec,
        out_shape=jax.ShapeDtypeStruct((H, Wd * C), jnp.float32),
        input_output_aliases={1: 0},
    )(sidx, ycopy, x2d, a_all, t_row)

    return y2d.reshape(N, H, Wd, C)


# speedup vs baseline: 22.6048x; 1.1067x over previous
import jax
import jax.numpy as jnp
from jax.experimental import pallas as pl
from jax.experimental.pallas import tpu as pltpu

_BS = 16
_C = 32
_BC = _BS * _C
_G = 16
_EPS = 1e-3


def _conv_body(sidx_ref, *refs):
    x_refs = refs[:_G]
    a_ref, t_ref, o_ref = refs[_G:]
    del sidx_ref
    pall = jnp.concatenate([r[...] for r in x_refs], axis=0)
    z = jnp.zeros((1, _BC), pall.dtype)
    up = jnp.concatenate([z, pall[:-1, :]], axis=0)
    dn = jnp.concatenate([pall[1:, :], z], axis=0)
    r = jax.lax.broadcasted_iota(jnp.int32, (_G * _BS, 1), 0) % _BS
    up = jnp.where(r != 0, up, 0.0)
    dn = jnp.where(r != _BS - 1, dn, 0.0)
    pc = jnp.concatenate([up, pall, dn], axis=1)
    q = jnp.dot(pc, a_ref[...], preferred_element_type=jnp.float32)
    q = jnp.maximum(q + t_ref[...], 0.0)
    o_ref[...] = q.reshape(_G, _BS, _BC)


def _band_body(starts_ref, bxs_ref, x_ref, d_ref, o_ref):
    o_ref[...] = x_ref[...]
    b = pl.program_id(0)

    def body(k, carry):
        o_ref[:, bxs_ref[k], :] = d_ref[k]
        return carry

    jax.lax.fori_loop(starts_ref[b], starts_ref[b + 1], body, 0)


def kernel(x, active_block_indices, bin_counts, W, b, gamma, beta,
           running_mean, running_var):
    del bin_counts
    N, H, Wd, C = x.shape
    gh = H // _BS
    gw = Wd // _BS
    nact = active_block_indices.shape[0]

    x2d = x.reshape(H, Wd * C)
    x3d = x.reshape(H, gw, _BC)

    by = (active_block_indices[:, 1] % gh).astype(jnp.int32)
    bx = (active_block_indices[:, 2] % gw).astype(jnp.int32)
    order = jnp.argsort(by)
    by_s = by[order]
    bx_s = bx[order]
    sidx = jnp.stack([by_s, bx_s])
    starts = jnp.searchsorted(by_s, jnp.arange(gh + 1)).astype(jnp.int32)

    s = gamma * jax.lax.rsqrt(running_var + _EPS)
    t = (b - running_mean) * s + beta
    wts = jnp.transpose(W, (2, 3, 1, 0)) * s

    a_rows = []
    for dy in range(3):
        a = jnp.zeros((_BC, _BC), jnp.float32)
        for dx in range(3):
            a = a + jnp.kron(jnp.eye(_BS, k=1 - dx, dtype=jnp.float32),
                             wts[dy, dx])
        a_rows.append(a)
    a_all = jnp.concatenate(a_rows, axis=0)
    t_row = jnp.tile(t, _BS).reshape(1, _BC)

    conv_spec = pltpu.PrefetchScalarGridSpec(
        num_scalar_prefetch=1,
        grid=(nact // _G,),
        in_specs=[
            *[pl.BlockSpec((_BS, _BC),
                           lambda g, s, j=j: (s[0, _G * g + j], s[1, _G * g + j]))
              for j in range(_G)],
            pl.BlockSpec((3 * _BC, _BC), lambda g, s: (0, 0)),
            pl.BlockSpec((1, _BC), lambda g, s: (0, 0)),
        ],
        out_specs=pl.BlockSpec((_G, _BS, _BC), lambda g, s: (g, 0, 0)),
    )
    dense = pl.pallas_call(
        _conv_body,
        grid_spec=conv_spec,
        out_shape=jax.ShapeDtypeStruct((nact, _BS, _BC), jnp.float32),
    )(sidx, *([x2d] * _G), a_all, t_row)

    band_spec = pltpu.PrefetchScalarGridSpec(
        num_scalar_prefetch=2,
        grid=(gh,),
        in_specs=[
            pl.BlockSpec((_BS, gw, _BC), lambda i, st, bxs: (i, 0, 0)),
            pl.BlockSpec((nact, _BS, _BC), lambda i, st, bxs: (0, 0, 0)),
        ],
        out_specs=pl.BlockSpec((_BS, gw, _BC), lambda i, st, bxs: (i, 0, 0)),
    )
    y3d = pl.pallas_call(
        _band_body,
        grid_spec=band_spec,
        out_shape=jax.ShapeDtypeStruct((H, gw, _BC), jnp.float32),
    )(starts, bx_s, x3d, dense)

    return y3d.reshape(N, H, Wd, C)
